# baseline (speedup 1.0000x reference)
import functools

import jax
import jax.numpy as jnp
from jax import lax
from jax.experimental import pallas as pl
from jax.experimental.pallas import tpu as pltpu
from jax.experimental.pallas import tpu_sc as plsc

SEQ = 200
BATCH = 4096
EMB = 64
LANES = 16
NC, NS = 2, 16
NW = NC * NS
BPW = BATCH // NW
NBUF = 2


def _sc_gather_sum(text, emb_table):
    mesh = plsc.VectorSubcoreMesh(core_axis_name="c", subcore_axis_name="s")

    @functools.partial(
        pl.kernel,
        mesh=mesh,
        out_type=jax.ShapeDtypeStruct((BATCH, EMB), jnp.float32),
        scratch_types=[
            pltpu.VMEM((SEQ, BPW), jnp.int32),
            pltpu.VMEM((BPW, EMB), jnp.float32),
            pltpu.VMEM((NBUF, BPW, EMB), jnp.float32),
            pltpu.SemaphoreType.DMA,
            pltpu.SemaphoreType.DMA,
        ],
        compiler_params=pltpu.CompilerParams(use_tc_tiling_on_sc=False),
    )
    def k(text_hbm, table_hbm, out_hbm, idx_v, acc_v, rows_v, sem0, sem1):
        wid = lax.axis_index("s") * NC + lax.axis_index("c")
        b0 = wid * BPW
        sems = (sem0, sem1)

        pltpu.sync_copy(text_hbm.at[:, pl.ds(b0, BPW)], idx_v)

        zero16 = jnp.zeros((LANES,), jnp.float32)

        def zbody(r, carry):
            for c in range(EMB // LANES):
                acc_v[r, pl.ds(c * LANES, LANES)] = zero16
            return carry

        lax.fori_loop(0, BPW, zbody, 0)

        def start_gather(s, buf):
            pltpu.make_async_copy(
                table_hbm.at[idx_v.at[s]], rows_v.at[buf], sems[buf]
            ).start()

        def wait_gather(buf):
            pltpu.make_async_copy(
                table_hbm.at[idx_v.at[0]], rows_v.at[buf], sems[buf]
            ).wait()

        def accumulate(buf):
            rows = rows_v.at[buf]

            def abody(r, carry):
                for c in range(EMB // LANES):
                    sl = pl.ds(c * LANES, LANES)
                    plsc.addupdate(acc_v.at[r, sl], rows[r, sl])
                return carry

            lax.fori_loop(0, BPW, abody, 0)

        for b in range(NBUF):
            start_gather(b, b)

        def mbody(g, carry):
            for b in range(NBUF):
                s = g * NBUF + b
                wait_gather(b)
                accumulate(b)

                @pl.when(s + NBUF < SEQ)
                def _():
                    start_gather(s + NBUF, b)

            return carry

        lax.fori_loop(0, SEQ // NBUF, mbody, 0)

        pltpu.sync_copy(acc_v, out_hbm.at[pl.ds(b0, BPW)])

    return k(text, emb_table)


def _tc_mlp(sums, W1, b1, W2, b2, Wf, bf):
    def body(s_ref, w1_ref, b1_ref, w2_ref, b2_ref, wf_ref, bf_ref, o_ref):
        x = s_ref[...] * (1.0 / SEQ)
        h = jnp.dot(x, w1_ref[...], preferred_element_type=jnp.float32)
        h = h + b1_ref[...]
        h = jnp.dot(h, w2_ref[...], preferred_element_type=jnp.float32)
        h = h + b2_ref[...]
        o = jnp.dot(h, wf_ref[...], preferred_element_type=jnp.float32)
        o_ref[...] = o + bf_ref[...]

    return pl.pallas_call(
        body,
        out_shape=jax.ShapeDtypeStruct((BATCH, Wf.shape[1]), jnp.float32),
    )(sums, W1, b1.reshape(1, -1), W2, b2.reshape(1, -1), Wf, bf.reshape(1, -1))


def kernel(text, emb_table, W1, b1, W2, b2, Wf, bf):
    sums = _sc_gather_sum(text, emb_table)
    return _tc_mlp(sums, W1, b1, W2, b2, Wf, bf)


# speedup vs baseline: 3.9866x; 3.9866x over previous
import functools

import jax
import jax.numpy as jnp
from jax import lax
from jax.experimental import pallas as pl
from jax.experimental.pallas import tpu as pltpu
from jax.experimental.pallas import tpu_sc as plsc

SEQ = 200
BATCH = 4096
EMB = 64
VOCAB = 1000000
PW = 16
PACK = 128 // PW
LANES = 16
NC, NS = 2, 16
NW = NC * NS
BPW = BATCH // NW
GRP = 4
NGRP = 2
BK = 16384
NBLK = (VOCAB + BK - 1) // BK
PROWS = NBLK * BK


def _tc_weights(W1, W2, Wf_pad):
    def body(w1_ref, w2_ref, wf_ref, o_ref):
        h = jnp.dot(w1_ref[...], w2_ref[...], preferred_element_type=jnp.float32)
        o_ref[...] = jnp.dot(h, wf_ref[...], preferred_element_type=jnp.float32)

    return pl.pallas_call(
        body,
        out_shape=jax.ShapeDtypeStruct((EMB, PW), jnp.float32),
    )(W1, W2, Wf_pad)


def _tc_project(tableT, Wc):
    sub = BK // PACK

    def body(x_ref, wc_ref, o_ref):
        pt = lax.dot_general(
            wc_ref[...], x_ref[...], (((0,), (0,)), ((), ())),
            preferred_element_type=jnp.float32,
        )
        v = jnp.concatenate(
            [pt[:, s * sub:(s + 1) * sub] for s in range(PACK)], axis=0
        )
        o_ref[...] = v.T

    return pl.pallas_call(
        body,
        grid=(NBLK,),
        in_specs=[
            pl.BlockSpec((EMB, BK), lambda i: (0, i)),
            pl.BlockSpec((EMB, PW), lambda i: (0, 0)),
        ],
        out_specs=pl.BlockSpec((sub, 128), lambda i: (i, 0)),
        out_shape=jax.ShapeDtypeStruct((NBLK * sub, 128), jnp.float32),
    )(tableT, Wc)


def _sc_gather_sum(text, ptable):
    mesh = plsc.VectorSubcoreMesh(core_axis_name="c", subcore_axis_name="s")

    @functools.partial(
        pl.kernel,
        mesh=mesh,
        out_type=jax.ShapeDtypeStruct((BATCH, PW), jnp.float32),
        scratch_types=[
            pltpu.VMEM((SEQ, BPW), jnp.int32),
            pltpu.VMEM((BPW, PW), jnp.float32),
            pltpu.VMEM((NGRP * GRP, BPW, PW), jnp.float32),
            pltpu.SemaphoreType.DMA,
            pltpu.SemaphoreType.DMA,
        ],
        compiler_params=pltpu.CompilerParams(use_tc_tiling_on_sc=False),
    )
    def k(text_hbm, table_hbm, out_hbm, idx_v, acc_v, rows_v, sem0, sem1):
        wid = lax.axis_index("s") * NC + lax.axis_index("c")
        b0 = wid * BPW
        sems = (sem0, sem1)

        pltpu.sync_copy(text_hbm.at[:, pl.ds(b0, BPW)], idx_v)

        @plsc.parallel_loop(0, SEQ, unroll=4)
        def _(r):
            for c in range(BPW // LANES):
                sl = pl.ds(c * LANES, LANES)
                v = idx_v[r, sl]
                blk = jnp.bitwise_and(v, jnp.int32(-BK))
                rr = jnp.bitwise_and(v, jnp.int32(BK // PACK - 1))
                ss = jnp.bitwise_and(
                    lax.shift_right_logical(v, jnp.int32(11)), jnp.int32(PACK - 1)
                )
                idx_v[r, sl] = blk + lax.shift_left(rr, jnp.int32(3)) + ss

        zero16 = jnp.zeros((LANES,), jnp.float32)

        @plsc.parallel_loop(0, BPW, unroll=8)
        def _(r):
            acc_v[r, pl.ds(0, LANES)] = zero16

        def start_gather(s, buf, grp):
            pltpu.make_async_copy(
                table_hbm.at[idx_v.at[s]], rows_v.at[buf], sems[grp]
            ).start()

        def wait_group(grp):
            for j in range(GRP):
                pltpu.make_async_copy(
                    table_hbm.at[idx_v.at[0]], rows_v.at[grp * GRP + j], sems[grp]
                ).wait()

        def accumulate_group(grp):
            b = grp * GRP

            @plsc.parallel_loop(0, BPW, unroll=4)
            def _(r):
                sl = pl.ds(0, LANES)
                v01 = rows_v[b, r, sl] + rows_v[b + 1, r, sl]
                v23 = rows_v[b + 2, r, sl] + rows_v[b + 3, r, sl]
                plsc.addupdate(acc_v.at[r, sl], v01 + v23)

        for grp in range(NGRP):
            for j in range(GRP):
                start_gather(grp * GRP + j, grp * GRP + j, grp)

        STRIDE = NGRP * GRP

        def mbody(g, carry):
            for grp in range(NGRP):
                base = g * STRIDE + grp * GRP
                wait_group(grp)
                accumulate_group(grp)
                for j in range(GRP):
                    s = base + STRIDE + j

                    @pl.when(s < SEQ)
                    def _():
                        start_gather(s, grp * GRP + j, grp)

            return carry

        lax.fori_loop(0, SEQ // STRIDE, mbody, 0)

        pltpu.sync_copy(acc_v, out_hbm.at[pl.ds(b0, BPW)])

    return k(text, ptable)


def _tc_tail(sums, W2, b1, b2, Wf, bf):
    def body(s_ref, w2_ref, b1_ref, b2_ref, wf_ref, bf_ref, o_ref):
        h = jnp.dot(b1_ref[...], w2_ref[...], preferred_element_type=jnp.float32)
        h = h + b2_ref[...]
        bias = jnp.dot(h, wf_ref[...], preferred_element_type=jnp.float32)
        bias = bias + bf_ref[...]
        o_ref[...] = s_ref[...][:, : Wf.shape[1]] * (1.0 / SEQ) + bias

    return pl.pallas_call(
        body,
        out_shape=jax.ShapeDtypeStruct((BATCH, Wf.shape[1]), jnp.float32),
    )(sums, W2, b1.reshape(1, -1), b2.reshape(1, -1), Wf, bf.reshape(1, -1))


def kernel(text, emb_table, W1, b1, W2, b2, Wf, bf):
    Wf_pad = jnp.pad(Wf, ((0, 0), (0, PW - Wf.shape[1])))
    Wc = _tc_weights(W1, W2, Wf_pad)
    packed = _tc_project(emb_table.T, Wc)
    sums = _sc_gather_sum(text, packed.reshape(PROWS, PW))
    return _tc_tail(sums, W2, b1, b2, Wf, bf)


# speedup vs baseline: 4.1502x; 1.0410x over previous
import functools

import jax
import jax.numpy as jnp
from jax import lax
from jax.experimental import pallas as pl
from jax.experimental.pallas import tpu as pltpu
from jax.experimental.pallas import tpu_sc as plsc

SEQ = 200
BATCH = 4096
EMB = 64
VOCAB = 1000000
OUT = 2
PW = 16
PACK = 128 // PW
LANES = 16
NC, NS = 2, 16
NW = NC * NS
BPW = BATCH // NW
GRP = 8
NGRP = 2
BK = 32768
SUB = BK // PACK
NBLK = (VOCAB + BK - 1) // BK
PROWS = NBLK * BK


def _tc_weights(W1, W2, Wf_pad, b1, b2, bf_pad):
    def body(w1_ref, w2_ref, wf_ref, b1_ref, b2_ref, bf_ref, wc_ref, bias_ref):
        h = jnp.dot(
            w1_ref[...] * (1.0 / SEQ), w2_ref[...],
            preferred_element_type=jnp.float32,
        )
        wc_ref[...] = jnp.dot(h, wf_ref[...], preferred_element_type=jnp.float32)
        hb = jnp.dot(b1_ref[...], w2_ref[...], preferred_element_type=jnp.float32)
        hb = hb + b2_ref[...]
        bias_ref[...] = (
            jnp.dot(hb, wf_ref[...], preferred_element_type=jnp.float32)
            + bf_ref[...]
        )

    return pl.pallas_call(
        body,
        out_shape=(
            jax.ShapeDtypeStruct((EMB, PW), jnp.float32),
            jax.ShapeDtypeStruct((1, PW), jnp.float32),
        ),
    )(W1, W2, Wf_pad, b1.reshape(1, -1), b2.reshape(1, -1), bf_pad)


def _tc_project(tableT, Wc):
    def body(x_ref, wc_ref, o_ref):
        pt = lax.dot_general(
            wc_ref[...], x_ref[...], (((0,), (0,)), ((), ())),
            preferred_element_type=jnp.float32,
        )
        v = jnp.concatenate(
            [pt[:, s * SUB:(s + 1) * SUB] for s in range(PACK)], axis=0
        )
        o_ref[...] = v.T

    return pl.pallas_call(
        body,
        grid=(NBLK,),
        in_specs=[
            pl.BlockSpec((EMB, BK), lambda i: (0, i)),
            pl.BlockSpec((EMB, PW), lambda i: (0, 0)),
        ],
        out_specs=pl.BlockSpec((SUB, 128), lambda i: (i, 0)),
        out_shape=jax.ShapeDtypeStruct((NBLK * SUB, 128), jnp.float32),
    )(tableT, Wc)


def _sc_gather_sum(text, ptable, bias):
    mesh = plsc.VectorSubcoreMesh(core_axis_name="c", subcore_axis_name="s")

    @functools.partial(
        pl.kernel,
        mesh=mesh,
        out_type=jax.ShapeDtypeStruct((BATCH, OUT), jnp.float32),
        scratch_types=[
            pltpu.VMEM((SEQ, BPW), jnp.int32),
            pltpu.VMEM((BPW, PW), jnp.float32),
            pltpu.VMEM((NGRP * GRP, BPW, PW), jnp.float32),
            pltpu.VMEM((1, PW), jnp.float32),
            pltpu.SemaphoreType.DMA,
            pltpu.SemaphoreType.DMA,
        ],
        compiler_params=pltpu.CompilerParams(use_tc_tiling_on_sc=False),
    )
    def k(text_hbm, table_hbm, bias_hbm, out_hbm,
          idx_v, acc_v, rows_v, bias_v, sem0, sem1):
        wid = lax.axis_index("s") * NC + lax.axis_index("c")
        b0 = wid * BPW
        sems = (sem0, sem1)

        pltpu.sync_copy(text_hbm.at[:, pl.ds(b0, BPW)], idx_v)
        pltpu.sync_copy(bias_hbm, bias_v)

        sh_s = SUB.bit_length() - 1
        sh_p = PACK.bit_length() - 1

        @plsc.parallel_loop(0, SEQ, unroll=4)
        def _(r):
            for c in range(BPW // LANES):
                sl = pl.ds(c * LANES, LANES)
                v = idx_v[r, sl]
                blk = jnp.bitwise_and(v, jnp.int32(-BK))
                rr = jnp.bitwise_and(v, jnp.int32(SUB - 1))
                ss = jnp.bitwise_and(
                    lax.shift_right_logical(v, jnp.int32(sh_s)),
                    jnp.int32(PACK - 1),
                )
                idx_v[r, sl] = blk + lax.shift_left(rr, jnp.int32(sh_p)) + ss

        zero16 = jnp.zeros((LANES,), jnp.float32)

        @plsc.parallel_loop(0, BPW, unroll=8)
        def _(r):
            acc_v[r, pl.ds(0, LANES)] = zero16

        def start_gather(s, buf, grp):
            pltpu.make_async_copy(
                table_hbm.at[idx_v.at[s]], rows_v.at[buf], sems[grp]
            ).start()

        def wait_group(grp):
            for j in range(GRP):
                pltpu.make_async_copy(
                    table_hbm.at[idx_v.at[0]], rows_v.at[grp * GRP + j], sems[grp]
                ).wait()

        def accumulate_group(grp):
            b = grp * GRP

            @plsc.parallel_loop(0, BPW, unroll=4)
            def _(r):
                sl = pl.ds(0, LANES)
                v01 = rows_v[b, r, sl] + rows_v[b + 1, r, sl]
                v23 = rows_v[b + 2, r, sl] + rows_v[b + 3, r, sl]
                v45 = rows_v[b + 4, r, sl] + rows_v[b + 5, r, sl]
                v67 = rows_v[b + 6, r, sl] + rows_v[b + 7, r, sl]
                plsc.addupdate(acc_v.at[r, sl], (v01 + v23) + (v45 + v67))

        for grp in range(NGRP):
            for j in range(GRP):
                start_gather(grp * GRP + j, grp * GRP + j, grp)

        STRIDE = NGRP * GRP

        def mbody(g, carry):
            for grp in range(NGRP):
                base = g * STRIDE + grp * GRP
                wait_group(grp)
                accumulate_group(grp)
                for j in range(GRP):
                    s = base + STRIDE + j

                    @pl.when(s < SEQ)
                    def _():
                        start_gather(s, grp * GRP + j, grp)

            return carry

        lax.fori_loop(0, SEQ // STRIDE, mbody, 0)

        if SEQ % STRIDE:
            wait_group(0)
            accumulate_group(0)

        bias = bias_v[0, pl.ds(0, LANES)]

        @plsc.parallel_loop(0, BPW, unroll=8)
        def _(r):
            sl = pl.ds(0, LANES)
            acc_v[r, sl] = acc_v[r, sl] + bias

        pltpu.sync_copy(acc_v.at[:, pl.ds(0, OUT)], out_hbm.at[pl.ds(b0, BPW)])

    return k(text, ptable, bias)


def kernel(text, emb_table, W1, b1, W2, b2, Wf, bf):
    Wf_pad = jnp.pad(Wf, ((0, 0), (0, PW - OUT)))
    bf_pad = jnp.pad(bf, (0, PW - OUT)).reshape(1, -1)
    Wc, bias = _tc_weights(W1, W2, Wf_pad, b1, b2, bf_pad)
    packed = _tc_project(emb_table.T, Wc)
    return _sc_gather_sum(text, packed.reshape(PROWS, PW), bias)


# speedup vs baseline: 4.2485x; 1.0237x over previous
import functools

import jax
import jax.numpy as jnp
from jax import lax
from jax.experimental import pallas as pl
from jax.experimental.pallas import tpu as pltpu
from jax.experimental.pallas import tpu_sc as plsc

SEQ = 200
BATCH = 4096
EMB = 64
VOCAB = 1000000
OUT = 2
PW = 16
PACK = 128 // PW
LANES = 16
NC, NS = 2, 16
NW = NC * NS
BPW = BATCH // NW
GRP = 4
NGRP = 2
BK = 32768
SUB = BK // PACK
NBLK = (VOCAB + BK - 1) // BK
PROWS = NBLK * BK


def _tc_weights(W1, W2, Wf_pad, b1, b2, bf_pad):
    def body(w1_ref, w2_ref, wf_ref, b1_ref, b2_ref, bf_ref, wc_ref, bias_ref):
        h = jnp.dot(
            w1_ref[...] * (1.0 / SEQ), w2_ref[...],
            preferred_element_type=jnp.float32,
        )
        wc_ref[...] = jnp.dot(h, wf_ref[...], preferred_element_type=jnp.float32)
        hb = jnp.dot(b1_ref[...], w2_ref[...], preferred_element_type=jnp.float32)
        hb = hb + b2_ref[...]
        bias_ref[...] = (
            jnp.dot(hb, wf_ref[...], preferred_element_type=jnp.float32)
            + bf_ref[...]
        )

    return pl.pallas_call(
        body,
        out_shape=(
            jax.ShapeDtypeStruct((EMB, PW), jnp.float32),
            jax.ShapeDtypeStruct((1, PW), jnp.float32),
        ),
    )(W1, W2, Wf_pad, b1.reshape(1, -1), b2.reshape(1, -1), bf_pad)


def _tc_project(tableT, Wc):
    def body(x_ref, wc_ref, o_ref):
        pt = lax.dot_general(
            wc_ref[...], x_ref[...], (((0,), (0,)), ((), ())),
            preferred_element_type=jnp.float32,
        )
        v = jnp.concatenate(
            [pt[:, s * SUB:(s + 1) * SUB] for s in range(PACK)], axis=0
        )
        o_ref[...] = v.T

    return pl.pallas_call(
        body,
        grid=(NBLK,),
        in_specs=[
            pl.BlockSpec((EMB, BK), lambda i: (0, i)),
            pl.BlockSpec((EMB, PW), lambda i: (0, 0)),
        ],
        out_specs=pl.BlockSpec((SUB, 128), lambda i: (i, 0)),
        out_shape=jax.ShapeDtypeStruct((NBLK * SUB, 128), jnp.float32),
    )(tableT, Wc)


def _sc_prep_idx(text):
    mesh = plsc.VectorSubcoreMesh(core_axis_name="c", subcore_axis_name="s")

    @functools.partial(
        pl.kernel,
        mesh=mesh,
        out_type=jax.ShapeDtypeStruct((NW, SEQ, BPW), jnp.int32),
        scratch_types=[
            pltpu.VMEM((SEQ, BPW), jnp.int32),
        ],
        compiler_params=pltpu.CompilerParams(use_tc_tiling_on_sc=False),
    )
    def k(text_hbm, out_hbm, idx_v):
        wid = lax.axis_index("s") * NC + lax.axis_index("c")
        b0 = wid * BPW

        pltpu.sync_copy(text_hbm.at[:, pl.ds(b0, BPW)], idx_v)

        sh_s = SUB.bit_length() - 1
        sh_p = PACK.bit_length() - 1

        @plsc.parallel_loop(0, SEQ, unroll=4)
        def _(r):
            for c in range(BPW // LANES):
                sl = pl.ds(c * LANES, LANES)
                v = idx_v[r, sl]
                blk = jnp.bitwise_and(v, jnp.int32(-BK))
                rr = jnp.bitwise_and(v, jnp.int32(SUB - 1))
                ss = jnp.bitwise_and(
                    lax.shift_right_logical(v, jnp.int32(sh_s)),
                    jnp.int32(PACK - 1),
                )
                idx_v[r, sl] = blk + lax.shift_left(rr, jnp.int32(sh_p)) + ss

        pltpu.sync_copy(idx_v, out_hbm.at[wid])

    return k(text)


def _sc_gather_sum(idx2, ptable, bias):
    mesh = plsc.VectorSubcoreMesh(core_axis_name="c", subcore_axis_name="s")

    @functools.partial(
        pl.kernel,
        mesh=mesh,
        out_type=jax.ShapeDtypeStruct((BATCH, OUT), jnp.float32),
        scratch_types=[
            pltpu.VMEM((SEQ, BPW), jnp.int32),
            pltpu.VMEM((BPW, PW), jnp.float32),
            pltpu.VMEM((NGRP * GRP, BPW, PW), jnp.float32),
            pltpu.VMEM((1, PW), jnp.float32),
            pltpu.SemaphoreType.DMA,
            pltpu.SemaphoreType.DMA,
        ],
        compiler_params=pltpu.CompilerParams(use_tc_tiling_on_sc=False),
    )
    def k(idx_hbm, table_hbm, bias_hbm, out_hbm,
          idx_v, acc_v, rows_v, bias_v, sem0, sem1):
        wid = lax.axis_index("s") * NC + lax.axis_index("c")
        b0 = wid * BPW
        sems = (sem0, sem1)

        pltpu.sync_copy(idx_hbm.at[wid], idx_v)
        pltpu.sync_copy(bias_hbm, bias_v)

        zero16 = jnp.zeros((LANES,), jnp.float32)

        @plsc.parallel_loop(0, BPW, unroll=8)
        def _(r):
            acc_v[r, pl.ds(0, LANES)] = zero16

        def start_gather(s, buf, grp):
            pltpu.make_async_copy(
                table_hbm.at[idx_v.at[s]], rows_v.at[buf], sems[grp]
            ).start()

        def wait_group(grp):
            for j in range(GRP):
                pltpu.make_async_copy(
                    table_hbm.at[idx_v.at[0]], rows_v.at[grp * GRP + j], sems[grp]
                ).wait()

        def accumulate_group(grp):
            b = grp * GRP

            @plsc.parallel_loop(0, BPW, unroll=4)
            def _(r):
                sl = pl.ds(0, LANES)
                v01 = rows_v[b, r, sl] + rows_v[b + 1, r, sl]
                v23 = rows_v[b + 2, r, sl] + rows_v[b + 3, r, sl]
                plsc.addupdate(acc_v.at[r, sl], v01 + v23)

        for grp in range(NGRP):
            for j in range(GRP):
                start_gather(grp * GRP + j, grp * GRP + j, grp)

        STRIDE = NGRP * GRP

        def mbody(g, carry):
            for grp in range(NGRP):
                base = g * STRIDE + grp * GRP
                wait_group(grp)
                accumulate_group(grp)
                for j in range(GRP):
                    s = base + STRIDE + j

                    @pl.when(s < SEQ)
                    def _():
                        start_gather(s, grp * GRP + j, grp)

            return carry

        lax.fori_loop(0, SEQ // STRIDE, mbody, 0)

        if SEQ % STRIDE:
            wait_group(0)
            accumulate_group(0)

        bias = bias_v[0, pl.ds(0, LANES)]

        @plsc.parallel_loop(0, BPW, unroll=8)
        def _(r):
            sl = pl.ds(0, LANES)
            acc_v[r, sl] = acc_v[r, sl] + bias

        pltpu.sync_copy(acc_v.at[:, pl.ds(0, OUT)], out_hbm.at[pl.ds(b0, BPW)])

    return k(idx2, ptable, bias)


def kernel(text, emb_table, W1, b1, W2, b2, Wf, bf):
    Wf_pad = jnp.pad(Wf, ((0, 0), (0, PW - OUT)))
    bf_pad = jnp.pad(bf, (0, PW - OUT)).reshape(1, -1)
    Wc, bias = _tc_weights(W1, W2, Wf_pad, b1, b2, bf_pad)
    idx2 = _sc_prep_idx(text)
    packed = _tc_project(emb_table.T, Wc)
    return _sc_gather_sum(idx2, packed.reshape(PROWS, PW), bias)
